# Initial kernel scaffold; baseline (speedup 1.0000x reference)
import jax
import jax.numpy as jnp
from jax.experimental import pallas as pl


def kernel(x, W_embed, W_masks):
    raise NotImplementedError("write your pallas kernel here")


# speedup vs baseline: 15.8701x; 15.8701x over previous
import functools

import jax
import jax.numpy as jnp
from jax import lax
from jax.experimental import pallas as pl
from jax.experimental.pallas import tpu as pltpu
from jax.experimental.pallas import tpu_sc as plsc


_ROWS_BLK = 8192
_ROWDOT_PAD = 1 << 20


def _rowdot_body(we_ref, wm_ref, out_ref):
    out_ref[...] = jnp.sum(we_ref[...] * wm_ref[...], axis=1)


def _rowdot(we, wm):
    n, d = we.shape
    grid = pl.cdiv(n, _ROWS_BLK)
    return pl.pallas_call(
        _rowdot_body,
        grid=(grid,),
        in_specs=[
            pl.BlockSpec((_ROWS_BLK, d), lambda i: (i, 0)),
            pl.BlockSpec((_ROWS_BLK, d), lambda i: (i, 0)),
        ],
        out_specs=pl.BlockSpec((_ROWS_BLK,), lambda i: (i,)),
        out_shape=jax.ShapeDtypeStruct((_ROWDOT_PAD,), jnp.float32),
    )(we, wm)


_CHUNK = 2048


@functools.cache
def _make_gather(n_idx):
    info = plsc.get_sparse_core_info()
    nc, ns = info.num_cores, info.num_subcores
    nw = nc * ns
    per_w = n_idx // nw
    n_ch = per_w // _CHUNK
    mesh = plsc.VectorSubcoreMesh(core_axis_name="c", subcore_axis_name="s")

    @functools.partial(
        pl.kernel,
        mesh=mesh,
        out_type=jax.ShapeDtypeStruct((n_idx,), jnp.float32),
        scratch_types=[
            pltpu.VMEM((_CHUNK,), jnp.int32),
            pltpu.VMEM((_CHUNK,), jnp.float32),
            pltpu.SemaphoreType.DMA,
        ],
    )
    def gather_k(rowdot_hbm, xf_hbm, out_hbm, idx_v, val_v, sem):
        wid = lax.axis_index("s") * nc + lax.axis_index("c")
        base = wid * per_w

        def body(i, carry):
            off = base + i * _CHUNK
            pltpu.sync_copy(xf_hbm.at[pl.ds(off, _CHUNK)], idx_v)
            pltpu.async_copy(rowdot_hbm.at[idx_v], val_v, sem).wait()
            pltpu.sync_copy(val_v, out_hbm.at[pl.ds(off, _CHUNK)])
            return carry

        lax.fori_loop(0, n_ch, body, 0)

    return gather_k


def kernel(x, W_embed, W_masks):
    rowdot = _rowdot(W_embed, W_masks)
    xf = x.reshape(-1)
    out = _make_gather(xf.shape[0])(rowdot, xf)
    return out.reshape(x.shape)


# speedup vs baseline: 17.2671x; 1.0880x over previous
import functools

import jax
import jax.numpy as jnp
from jax import lax
from jax.experimental import pallas as pl
from jax.experimental.pallas import tpu as pltpu
from jax.experimental.pallas import tpu_sc as plsc


_FLAT_BLK = 1 << 19
_LANES = 4096
_ROWDOT_PAD = 1 << 20


def _rowdot_body(we_ref, wm_ref, sel_ref, out_ref):
    r = _FLAT_BLK // _LANES
    p = (we_ref[...] * wm_ref[...]).reshape(r, _LANES)
    out_ref[...] = jax.lax.dot_general(
        p, sel_ref[...], (((1,), (0,)), ((), ())),
        preferred_element_type=jnp.float32)


def _rowdot(we, wm):
    n, d = we.shape
    ef = we.reshape(-1)
    mf = wm.reshape(-1)
    total = n * d
    grid = pl.cdiv(total, _FLAT_BLK)
    out_rows = _FLAT_BLK // _LANES
    sel = (jnp.arange(_LANES, dtype=jnp.int32)[:, None] // 32
           == jnp.arange(128, dtype=jnp.int32)[None, :]).astype(jnp.float32)
    out2d = pl.pallas_call(
        _rowdot_body,
        grid=(grid,),
        in_specs=[
            pl.BlockSpec((_FLAT_BLK,), lambda i: (i,)),
            pl.BlockSpec((_FLAT_BLK,), lambda i: (i,)),
            pl.BlockSpec((_LANES, 128), lambda i: (0, 0)),
        ],
        out_specs=pl.BlockSpec((out_rows, 128), lambda i: (i, 0)),
        out_shape=jax.ShapeDtypeStruct((_ROWDOT_PAD // 128, 128), jnp.float32),
    )(ef, mf, sel)
    return out2d.reshape(-1)


_CHUNK = 12800


@functools.cache
def _make_gather(n_idx):
    info = plsc.get_sparse_core_info()
    nc, ns = info.num_cores, info.num_subcores
    nw = nc * ns
    per_w = n_idx // nw
    n_ch = per_w // _CHUNK
    mesh = plsc.VectorSubcoreMesh(core_axis_name="c", subcore_axis_name="s")

    @functools.partial(
        pl.kernel,
        mesh=mesh,
        out_type=jax.ShapeDtypeStruct((n_idx,), jnp.float32),
        scratch_types=[
            pltpu.VMEM((_CHUNK,), jnp.int32),
            pltpu.VMEM((_CHUNK,), jnp.float32),
            pltpu.SemaphoreType.DMA,
        ],
    )
    def gather_k(rowdot_hbm, xf_hbm, out_hbm, idx_v, val_v, sem):
        wid = lax.axis_index("s") * nc + lax.axis_index("c")
        base = wid * per_w

        def body(i, carry):
            off = base + i * _CHUNK
            pltpu.sync_copy(xf_hbm.at[pl.ds(off, _CHUNK)], idx_v)
            pltpu.async_copy(rowdot_hbm.at[idx_v], val_v, sem).wait()
            pltpu.sync_copy(val_v, out_hbm.at[pl.ds(off, _CHUNK)])
            return carry

        lax.fori_loop(0, n_ch, body, 0)

    return gather_k


def kernel(x, W_embed, W_masks):
    rowdot = _rowdot(W_embed, W_masks)
    xf = x.reshape(-1)
    out = _make_gather(xf.shape[0])(rowdot, xf)
    return out.reshape(x.shape)


# speedup vs baseline: 18.7602x; 1.0865x over previous
import functools

import jax
import jax.numpy as jnp
from jax import lax
from jax.experimental import pallas as pl
from jax.experimental.pallas import tpu as pltpu
from jax.experimental.pallas import tpu_sc as plsc


_ROWS_BLK = 8192


def _rowdot_body(we_ref, wm_ref, out_ref):
    p = we_ref[...] * wm_ref[...]
    ones = jnp.ones((1, p.shape[1]), jnp.float32)
    s = jax.lax.dot_general(
        ones, p, (((1,), (1,)), ((), ())),
        preferred_element_type=jnp.float32)
    out_ref[...] = s.reshape(1, 1, -1)


def _rowdot(we, wm):
    n, d = we.shape
    grid = pl.cdiv(n, _ROWS_BLK)
    out2d = pl.pallas_call(
        _rowdot_body,
        grid=(grid,),
        in_specs=[
            pl.BlockSpec((_ROWS_BLK, d), lambda i: (i, 0)),
            pl.BlockSpec((_ROWS_BLK, d), lambda i: (i, 0)),
        ],
        out_specs=pl.BlockSpec((1, 1, _ROWS_BLK), lambda i: (i, 0, 0)),
        out_shape=jax.ShapeDtypeStruct((grid, 1, _ROWS_BLK), jnp.float32),
    )(we, wm)
    return out2d.reshape(-1)


_CHUNK = 12800


@functools.cache
def _make_gather(n_idx):
    info = plsc.get_sparse_core_info()
    nc, ns = info.num_cores, info.num_subcores
    nw = nc * ns
    per_w = n_idx // nw
    n_ch = per_w // _CHUNK
    mesh = plsc.VectorSubcoreMesh(core_axis_name="c", subcore_axis_name="s")

    @functools.partial(
        pl.kernel,
        mesh=mesh,
        out_type=jax.ShapeDtypeStruct((n_idx,), jnp.float32),
        scratch_types=[
            pltpu.VMEM((_CHUNK,), jnp.int32),
            pltpu.VMEM((_CHUNK,), jnp.float32),
            pltpu.SemaphoreType.DMA,
        ],
    )
    def gather_k(rowdot_hbm, xf_hbm, out_hbm, idx_v, val_v, sem):
        wid = lax.axis_index("s") * nc + lax.axis_index("c")
        base = wid * per_w

        def body(i, carry):
            off = base + i * _CHUNK
            pltpu.sync_copy(xf_hbm.at[pl.ds(off, _CHUNK)], idx_v)
            pltpu.async_copy(rowdot_hbm.at[idx_v], val_v, sem).wait()
            pltpu.sync_copy(val_v, out_hbm.at[pl.ds(off, _CHUNK)])
            return carry

        lax.fori_loop(0, n_ch, body, 0)

    return gather_k


def kernel(x, W_embed, W_masks):
    rowdot = _rowdot(W_embed, W_masks)
    xf = x.reshape(-1)
    out = _make_gather(xf.shape[0])(rowdot, xf)
    return out.reshape(x.shape)
